# baseline (speedup 1.0000x reference)
import functools

import jax
import jax.numpy as jnp
from jax import lax
from jax.experimental import pallas as pl
from jax.experimental.pallas import tpu as pltpu
from jax.experimental.pallas import tpu_sc as plsc

B, N, D, K, HID = 8, 2048, 64, 16, 12
EPS = 1e-5
TW = 2 * D
BN1 = 512
BN3 = 512
NB1 = N // BN1

NC, NS = 2, 16
NW = NC * NS
L = 16
R = K * B * N
RPW = R // NW
CH = 128
NCH = RPW // CH

_INF = 3.0e38


def _stage1_body(pos_ref, post_ref, x_ref, wkT_ref, wvT_ref, a1T_ref,
                 tab_ref, idx_ref):
    b = pl.program_id(0)
    pb = pos_ref[0]
    pT = post_ref[0]
    xb = x_ref[0]

    xk = jnp.dot(xb, wkT_ref[...], preferred_element_type=jnp.float32)
    tab_ref[:, 0:D] = jnp.dot(xk, a1T_ref[...], preferred_element_type=jnp.float32)
    tab_ref[:, D:TW] = jnp.dot(xb, wvT_ref[...], preferred_element_type=jnp.float32)

    sqi = jnp.sum(pb * pb, axis=1, keepdims=True)
    sqj = jnp.sum(pT * pT, axis=0, keepdims=True)
    pbh = pb.astype(jnp.bfloat16).astype(jnp.float32)
    pTh = pT.astype(jnp.bfloat16).astype(jnp.float32)
    dot = (pbh[:, 0:1] * pTh[0:1, :]
           + pbh[:, 1:2] * pTh[1:2, :]
           + pbh[:, 2:3] * pTh[2:3, :])
    dw = (sqi + sqj) - 2.0 * dot

    iota = lax.broadcasted_iota(jnp.int32, (BN1, N), 1).astype(jnp.float32)
    cols = []
    for _ in range(K):
        m = jnp.min(dw, axis=1, keepdims=True)
        cand = jnp.where(dw == m, iota, _INF)
        sel = jnp.min(cand, axis=1, keepdims=True)
        dw = jnp.where(cand == sel, _INF, dw)
        cols.append(sel)
    gi = jnp.concatenate(cols, axis=1).astype(jnp.int32) + b * N
    idx_ref[...] = gi


def _stage3_body(x_ref, pos_ref, g_ref, p_ref, wqT_ref, pw1T_ref, pb1_ref,
                 pg_ref, pbeta_ref, pw2T_ref, pb2_ref, a1T_ref, ab1_ref,
                 ag_ref, abeta_ref, a2T_ref, ab2_ref, o_ref):
    f32 = jnp.float32
    xb = x_ref[...]
    pb = pos_ref[...]
    a1T = a1T_ref[...]
    a2T = a2T_ref[...]
    pw2T = pw2T_ref[...]
    qa = jnp.dot(xb, jnp.dot(wqT_ref[...], a1T, preferred_element_type=f32),
                 preferred_element_type=f32)
    pw2a = jnp.dot(pw2T, a1T, preferred_element_type=f32)
    cb = jnp.dot(pb2_ref[...], a1T, preferred_element_type=f32) + ab1_ref[...]
    w1h = pw1T_ref[...].astype(jnp.bfloat16).astype(f32)
    M12 = jnp.full((HID, HID), 1.0 / HID, f32)
    M64 = jnp.full((D, D), 1.0 / D, f32)

    GF = g_ref[...].reshape(K * BN3, TW)
    KA = GF[:, 0:D]
    V = GF[:, D:TW]
    PF = p_ref[...].reshape(K * BN3, 4)
    PBT = jnp.concatenate([pb] * K, axis=0)
    QAT = jnp.concatenate([qa] * K, axis=0)
    pr = PBT - PF[:, 0:3]
    prh = pr.astype(jnp.bfloat16).astype(f32)
    H = (prh[:, 0:1] * w1h[0:1, :]
         + prh[:, 1:2] * w1h[1:2, :]
         + prh[:, 2:3] * w1h[2:3, :]) + pb1_ref[...]
    MU = jnp.dot(H, M12, preferred_element_type=f32)
    T = H - MU
    VAR = jnp.dot(T * T, M12, preferred_element_type=f32)
    HN = T * lax.rsqrt(VAR + EPS) * pg_ref[...] + pbeta_ref[...]
    RL = jnp.maximum(HN, 0.0)
    PE = jnp.dot(RL, pw2T, preferred_element_type=f32) + pb2_ref[...]
    W1L = QAT - KA + jnp.dot(RL, pw2a, preferred_element_type=f32) + cb
    MU2 = jnp.dot(W1L, M64, preferred_element_type=f32)
    T2 = W1L - MU2
    VAR2 = jnp.dot(T2 * T2, M64, preferred_element_type=f32)
    WN = T2 * lax.rsqrt(VAR2 + EPS) * ag_ref[...] + abeta_ref[...]
    WR = jnp.maximum(WN, 0.0)
    LOGIT = jnp.dot(WR, a2T, preferred_element_type=f32) + ab2_ref[...]
    VAL = V + PE

    L3 = LOGIT.reshape(K, BN3, D)
    V3 = VAL.reshape(K, BN3, D)
    m = L3[0]
    for k in range(1, K):
        m = jnp.maximum(m, L3[k])
    s = None
    acc = None
    for k in range(K):
        e = jnp.exp(L3[k] - m)
        s = e if s is None else s + e
        ev = e * V3[k]
        acc = ev if acc is None else acc + ev
    o_ref[...] = acc / s


def _sc_gather_body(tab_hbm, gidx_hbm, pos4_hbm, out_hbm, pout_hbm,
                    idx0_v, idx1_v, rows0_v, rows1_v, ptab_v, prow0_v,
                    prow1_v, sem0, sem1):
    wid = lax.axis_index("s") * NC + lax.axis_index("c")
    pltpu.sync_copy(pos4_hbm, ptab_v)
    lane = lax.broadcasted_iota(jnp.int32, (L,), 0)

    def pos_gather(idx_v, prow_v):
        for i in range(CH // L):
            iv = idx_v[pl.ds(i * L, L)] * 4
            for c in range(3):
                px = plsc.load_gather(ptab_v, [iv + c])
                plsc.store_scatter(prow_v, [lane * 4 + (i * 4 * L + c)], px)

    def body(u, carry):
        t0 = 2 * u
        t1 = t0 + 1
        b0 = wid * RPW + t0 * CH
        b1 = wid * RPW + t1 * CH
        pltpu.sync_copy(gidx_hbm.at[pl.ds(b0, CH)], idx0_v)
        cp0 = pltpu.async_copy(tab_hbm.at[idx0_v], rows0_v, sem0)
        pltpu.sync_copy(gidx_hbm.at[pl.ds(b1, CH)], idx1_v)
        cp1 = pltpu.async_copy(tab_hbm.at[idx1_v], rows1_v, sem1)
        pos_gather(idx0_v, prow0_v)
        cp0.wait()
        pltpu.sync_copy(rows0_v, out_hbm.at[pl.ds(b0, CH)])
        pltpu.sync_copy(prow0_v, pout_hbm.at[pl.ds(b0 * 4, CH * 4)])
        pos_gather(idx1_v, prow1_v)
        cp1.wait()
        pltpu.sync_copy(rows1_v, out_hbm.at[pl.ds(b1, CH)])
        pltpu.sync_copy(prow1_v, pout_hbm.at[pl.ds(b1 * 4, CH * 4)])
        return carry

    lax.fori_loop(0, NCH // 2, body, 0)


def _stage1_call(pos, posT, x, WkT, WvT, a1T):
    return pl.pallas_call(
        _stage1_body,
        grid=(B, NB1),
        in_specs=[
            pl.BlockSpec((1, BN1, 3), lambda b, i: (b, i, 0)),
            pl.BlockSpec((1, 3, N), lambda b, i: (b, 0, 0)),
            pl.BlockSpec((1, BN1, D), lambda b, i: (b, i, 0)),
            pl.BlockSpec((D, D), lambda b, i: (0, 0)),
            pl.BlockSpec((D, D), lambda b, i: (0, 0)),
            pl.BlockSpec((D, D), lambda b, i: (0, 0)),
        ],
        out_specs=[
            pl.BlockSpec((BN1, TW), lambda b, i: (b * NB1 + i, 0)),
            pl.BlockSpec((BN1, K), lambda b, i: (b * NB1 + i, 0)),
        ],
        out_shape=[
            jax.ShapeDtypeStruct((B * N, TW), jnp.float32),
            jax.ShapeDtypeStruct((B * N, K), jnp.int32),
        ],
    )(pos, posT, x, WkT, WvT, a1T)


def _stage2_call(tab, gidx_kmajor, pos4):
    run = pl.kernel(
        _sc_gather_body,
        out_type=(jax.ShapeDtypeStruct((R, TW), jnp.float32),
                  jax.ShapeDtypeStruct((R * 4,), jnp.float32)),
        mesh=plsc.VectorSubcoreMesh(core_axis_name="c", subcore_axis_name="s",
                                    num_cores=NC, num_subcores=NS),
        scratch_types=[
            pltpu.VMEM((CH,), jnp.int32),
            pltpu.VMEM((CH,), jnp.int32),
            pltpu.VMEM((CH, TW), jnp.float32),
            pltpu.VMEM((CH, TW), jnp.float32),
            pltpu.VMEM((B * N * 4,), jnp.float32),
            pltpu.VMEM((CH * 4,), jnp.float32),
            pltpu.VMEM((CH * 4,), jnp.float32),
            pltpu.SemaphoreType.DMA,
            pltpu.SemaphoreType.DMA,
        ],
        compiler_params=pltpu.CompilerParams(needs_layout_passes=False),
    )
    return run(tab, gidx_kmajor, pos4)


def _stage3_call(x2, pos2, G, P, WqT, pw1T, pb1, pg, pbeta, pw2T, pb2,
                 a1T, ab1, ag, abeta, a2T, ab2):
    nsteps = (B * N) // BN3
    full = lambda r, c: pl.BlockSpec((r, c), lambda i: (0, 0))
    return pl.pallas_call(
        _stage3_body,
        grid=(nsteps,),
        in_specs=[
            pl.BlockSpec((BN3, D), lambda i: (i, 0)),
            pl.BlockSpec((BN3, 3), lambda i: (i, 0)),
            pl.BlockSpec((K, BN3, TW), lambda i: (0, i, 0)),
            pl.BlockSpec((K, BN3, 4), lambda i: (0, i, 0)),
            full(D, D), full(3, HID), full(1, HID), full(1, HID),
            full(1, HID), full(HID, D), full(1, D), full(D, D), full(1, D),
            full(1, D), full(1, D), full(D, D), full(1, D),
        ],
        out_specs=pl.BlockSpec((BN3, D), lambda i: (i, 0)),
        out_shape=jax.ShapeDtypeStruct((B * N, D), jnp.float32),
    )(x2, pos2, G, P, WqT, pw1T, pb1, pg, pbeta, pw2T, pb2,
      a1T, ab1, ag, abeta, a2T, ab2)


def kernel(x, pos, Wq, Wk, Wv, pos_w1, pos_b1, pos_g, pos_beta, pos_w2,
           pos_b2, attn_w1, attn_b1, attn_g, attn_beta, attn_w2, attn_b2):
    posT = jnp.swapaxes(pos, 1, 2)
    tab, gidx = _stage1_call(pos, posT, x, Wk.T, Wv.T, attn_w1.T)
    gidx_kmajor = gidx.T.reshape(R)
    pos4 = jnp.pad(pos.reshape(B * N, 3), ((0, 0), (0, 1))).reshape(B * N * 4)
    G, P = _stage2_call(tab, gidx_kmajor, pos4)
    G = G.reshape(K, B * N, TW)
    P = P.reshape(K, B * N, 4)
    agg = _stage3_call(
        x.reshape(B * N, D), pos.reshape(B * N, 3), G, P,
        Wq.T, pos_w1.T, pos_b1.reshape(1, HID), pos_g.reshape(1, HID),
        pos_beta.reshape(1, HID), pos_w2.T, pos_b2.reshape(1, D),
        attn_w1.T, attn_b1.reshape(1, D), attn_g.reshape(1, D),
        attn_beta.reshape(1, D), attn_w2.T, attn_b2.reshape(1, D))
    return agg.reshape(B, N, D)


# speedup vs baseline: 1.0377x; 1.0377x over previous
import functools

import jax
import jax.numpy as jnp
from jax import lax
from jax.experimental import pallas as pl
from jax.experimental.pallas import tpu as pltpu
from jax.experimental.pallas import tpu_sc as plsc

B, N, D, K, HID = 8, 2048, 64, 16, 12
EPS = 1e-5
TW = 2 * D
BN1 = 512
BN3 = 512
NB1 = N // BN1

NC, NS = 2, 16
NW = NC * NS
L = 16
R = K * B * N
RPW = R // NW
CH = 128
NCH = RPW // CH

_INF = 3.0e38


def _stage1_body(pos_ref, post_ref, x_ref, wkT_ref, wvT_ref, a1T_ref,
                 tab_ref, idx_ref):
    b = pl.program_id(0)
    pb = pos_ref[0]
    pT = post_ref[0]
    xb = x_ref[0]

    xk = jnp.dot(xb, wkT_ref[...], preferred_element_type=jnp.float32)
    tab_ref[:, 0:D] = jnp.dot(xk, a1T_ref[...], preferred_element_type=jnp.float32)
    tab_ref[:, D:TW] = jnp.dot(xb, wvT_ref[...], preferred_element_type=jnp.float32)

    sqi = jnp.sum(pb * pb, axis=1, keepdims=True)
    sqj = jnp.sum(pT * pT, axis=0, keepdims=True)
    pbh = pb.astype(jnp.bfloat16).astype(jnp.float32)
    pTh = pT.astype(jnp.bfloat16).astype(jnp.float32)
    dot = (pbh[:, 0:1] * pTh[0:1, :]
           + pbh[:, 1:2] * pTh[1:2, :]
           + pbh[:, 2:3] * pTh[2:3, :])
    dw = (sqi + sqj) - 2.0 * dot

    iota = lax.broadcasted_iota(jnp.int32, (BN1, N), 1)
    cols = []
    for _ in range(K):
        sel = jnp.argmin(dw, axis=1).astype(jnp.int32).reshape(BN1, 1)
        dw = jnp.where(iota == sel, _INF, dw)
        cols.append(sel)
    gi = jnp.concatenate(cols, axis=1) + b * N
    idx_ref[...] = gi


def _stage3_body(x_ref, pos_ref, g_ref, p_ref, wqT_ref, pw1T_ref, pb1_ref,
                 pg_ref, pbeta_ref, pw2T_ref, pb2_ref, a1T_ref, ab1_ref,
                 ag_ref, abeta_ref, a2T_ref, ab2_ref, o_ref):
    f32 = jnp.float32
    xb = x_ref[...]
    pb = pos_ref[...]
    a1T = a1T_ref[...]
    a2T = a2T_ref[...]
    pw2T = pw2T_ref[...]
    qa = jnp.dot(xb, jnp.dot(wqT_ref[...], a1T, preferred_element_type=f32),
                 preferred_element_type=f32)
    pw2a = jnp.dot(pw2T, a1T, preferred_element_type=f32)
    cb = jnp.dot(pb2_ref[...], a1T, preferred_element_type=f32) + ab1_ref[...]
    w1h = pw1T_ref[...].astype(jnp.bfloat16).astype(f32)
    M12 = jnp.full((HID, HID), 1.0 / HID, f32)
    M64 = jnp.full((D, D), 1.0 / D, f32)

    GF = g_ref[...].reshape(K * BN3, TW)
    KA = GF[:, 0:D]
    V = GF[:, D:TW]
    PF = p_ref[...].reshape(K * BN3, 4)
    PBT = jnp.concatenate([pb] * K, axis=0)
    QAT = jnp.concatenate([qa] * K, axis=0)
    pr = PBT - PF[:, 0:3]
    prh = pr.astype(jnp.bfloat16).astype(f32)
    H = (prh[:, 0:1] * w1h[0:1, :]
         + prh[:, 1:2] * w1h[1:2, :]
         + prh[:, 2:3] * w1h[2:3, :]) + pb1_ref[...]
    MU = jnp.dot(H, M12, preferred_element_type=f32)
    T = H - MU
    VAR = jnp.dot(T * T, M12, preferred_element_type=f32)
    HN = T * lax.rsqrt(VAR + EPS) * pg_ref[...] + pbeta_ref[...]
    RL = jnp.maximum(HN, 0.0)
    PE = jnp.dot(RL, pw2T, preferred_element_type=f32) + pb2_ref[...]
    W1L = QAT - KA + jnp.dot(RL, pw2a, preferred_element_type=f32) + cb
    MU2 = jnp.dot(W1L, M64, preferred_element_type=f32)
    T2 = W1L - MU2
    VAR2 = jnp.dot(T2 * T2, M64, preferred_element_type=f32)
    WN = T2 * lax.rsqrt(VAR2 + EPS) * ag_ref[...] + abeta_ref[...]
    WR = jnp.maximum(WN, 0.0)
    LOGIT = jnp.dot(WR, a2T, preferred_element_type=f32)
    VAL = V + PE

    L3 = LOGIT.reshape(K, BN3, D)
    V3 = VAL.reshape(K, BN3, D)
    m = L3[0]
    for k in range(1, K):
        m = jnp.maximum(m, L3[k])
    s = None
    acc = None
    for k in range(K):
        e = jnp.exp(L3[k] - m)
        s = e if s is None else s + e
        ev = e * V3[k]
        acc = ev if acc is None else acc + ev
    o_ref[...] = acc / s


def _sc_gather_body(tab_hbm, gidx_hbm, pos4_hbm, out_hbm, pout_hbm,
                    idx0_v, idx1_v, rows0_v, rows1_v, ptab_v, prow0_v,
                    prow1_v, sem0, sem1):
    wid = lax.axis_index("s") * NC + lax.axis_index("c")
    pltpu.sync_copy(pos4_hbm, ptab_v)
    lane = lax.broadcasted_iota(jnp.int32, (L,), 0)

    def pos_gather(idx_v, prow_v):
        for i in range(CH // L):
            iv = idx_v[pl.ds(i * L, L)] * 4
            for c in range(3):
                px = plsc.load_gather(ptab_v, [iv + c])
                plsc.store_scatter(prow_v, [lane * 4 + (i * 4 * L + c)], px)

    def body(u, carry):
        t0 = 2 * u
        t1 = t0 + 1
        b0 = wid * RPW + t0 * CH
        b1 = wid * RPW + t1 * CH
        pltpu.sync_copy(gidx_hbm.at[pl.ds(b0, CH)], idx0_v)
        cp0 = pltpu.async_copy(tab_hbm.at[idx0_v], rows0_v, sem0)
        pltpu.sync_copy(gidx_hbm.at[pl.ds(b1, CH)], idx1_v)
        cp1 = pltpu.async_copy(tab_hbm.at[idx1_v], rows1_v, sem1)
        pos_gather(idx0_v, prow0_v)
        cp0.wait()
        pltpu.sync_copy(rows0_v, out_hbm.at[pl.ds(b0, CH)])
        pltpu.sync_copy(prow0_v, pout_hbm.at[pl.ds(b0 * 4, CH * 4)])
        pos_gather(idx1_v, prow1_v)
        cp1.wait()
        pltpu.sync_copy(rows1_v, out_hbm.at[pl.ds(b1, CH)])
        pltpu.sync_copy(prow1_v, pout_hbm.at[pl.ds(b1 * 4, CH * 4)])
        return carry

    lax.fori_loop(0, NCH // 2, body, 0)


def _stage1_call(pos, posT, x, WkT, WvT, a1T):
    return pl.pallas_call(
        _stage1_body,
        grid=(B, NB1),
        in_specs=[
            pl.BlockSpec((1, BN1, 3), lambda b, i: (b, i, 0)),
            pl.BlockSpec((1, 3, N), lambda b, i: (b, 0, 0)),
            pl.BlockSpec((1, BN1, D), lambda b, i: (b, i, 0)),
            pl.BlockSpec((D, D), lambda b, i: (0, 0)),
            pl.BlockSpec((D, D), lambda b, i: (0, 0)),
            pl.BlockSpec((D, D), lambda b, i: (0, 0)),
        ],
        out_specs=[
            pl.BlockSpec((BN1, TW), lambda b, i: (b * NB1 + i, 0)),
            pl.BlockSpec((BN1, K), lambda b, i: (b * NB1 + i, 0)),
        ],
        out_shape=[
            jax.ShapeDtypeStruct((B * N, TW), jnp.float32),
            jax.ShapeDtypeStruct((B * N, K), jnp.int32),
        ],
    )(pos, posT, x, WkT, WvT, a1T)


def _stage2_call(tab, gidx_kmajor, pos4):
    run = pl.kernel(
        _sc_gather_body,
        out_type=(jax.ShapeDtypeStruct((R, TW), jnp.float32),
                  jax.ShapeDtypeStruct((R * 4,), jnp.float32)),
        mesh=plsc.VectorSubcoreMesh(core_axis_name="c", subcore_axis_name="s",
                                    num_cores=NC, num_subcores=NS),
        scratch_types=[
            pltpu.VMEM((CH,), jnp.int32),
            pltpu.VMEM((CH,), jnp.int32),
            pltpu.VMEM((CH, TW), jnp.float32),
            pltpu.VMEM((CH, TW), jnp.float32),
            pltpu.VMEM((B * N * 4,), jnp.float32),
            pltpu.VMEM((CH * 4,), jnp.float32),
            pltpu.VMEM((CH * 4,), jnp.float32),
            pltpu.SemaphoreType.DMA,
            pltpu.SemaphoreType.DMA,
        ],
        compiler_params=pltpu.CompilerParams(needs_layout_passes=False),
    )
    return run(tab, gidx_kmajor, pos4)


def _stage3_call(x2, pos2, G, P, WqT, pw1T, pb1, pg, pbeta, pw2T, pb2,
                 a1T, ab1, ag, abeta, a2T, ab2):
    nsteps = (B * N) // BN3
    full = lambda r, c: pl.BlockSpec((r, c), lambda i: (0, 0))
    return pl.pallas_call(
        _stage3_body,
        grid=(nsteps,),
        in_specs=[
            pl.BlockSpec((BN3, D), lambda i: (i, 0)),
            pl.BlockSpec((BN3, 3), lambda i: (i, 0)),
            pl.BlockSpec((K, BN3, TW), lambda i: (0, i, 0)),
            pl.BlockSpec((K, BN3, 4), lambda i: (0, i, 0)),
            full(D, D), full(3, HID), full(1, HID), full(1, HID),
            full(1, HID), full(HID, D), full(1, D), full(D, D), full(1, D),
            full(1, D), full(1, D), full(D, D), full(1, D),
        ],
        out_specs=pl.BlockSpec((BN3, D), lambda i: (i, 0)),
        out_shape=jax.ShapeDtypeStruct((B * N, D), jnp.float32),
    )(x2, pos2, G, P, WqT, pw1T, pb1, pg, pbeta, pw2T, pb2,
      a1T, ab1, ag, abeta, a2T, ab2)


def kernel(x, pos, Wq, Wk, Wv, pos_w1, pos_b1, pos_g, pos_beta, pos_w2,
           pos_b2, attn_w1, attn_b1, attn_g, attn_beta, attn_w2, attn_b2):
    posT = jnp.swapaxes(pos, 1, 2)
    tab, gidx = _stage1_call(pos, posT, x, Wk.T, Wv.T, attn_w1.T)
    gidx_kmajor = gidx.T.reshape(R)
    pos4 = jnp.pad(pos.reshape(B * N, 3), ((0, 0), (0, 1))).reshape(B * N * 4)
    G, P = _stage2_call(tab, gidx_kmajor, pos4)
    G = G.reshape(K, B * N, TW)
    P = P.reshape(K, B * N, 4)
    agg = _stage3_call(
        x.reshape(B * N, D), pos.reshape(B * N, 3), G, P,
        Wq.T, pos_w1.T, pos_b1.reshape(1, HID), pos_g.reshape(1, HID),
        pos_beta.reshape(1, HID), pos_w2.T, pos_b2.reshape(1, D),
        attn_w1.T, attn_b1.reshape(1, D), attn_g.reshape(1, D),
        attn_beta.reshape(1, D), attn_w2.T, attn_b2.reshape(1, D))
    return agg.reshape(B, N, D)


# speedup vs baseline: 1.0455x; 1.0075x over previous
import functools

import jax
import jax.numpy as jnp
from jax import lax
from jax.experimental import pallas as pl
from jax.experimental.pallas import tpu as pltpu
from jax.experimental.pallas import tpu_sc as plsc

B, N, D, K, HID = 8, 2048, 64, 16, 12
EPS = 1e-5
TW = 2 * D
BN1 = 512
BN3 = 512
NB1 = N // BN1

NC, NS = 2, 16
NW = NC * NS
L = 16
R = K * B * N
RPW = R // NW
CH = 128
NCH = RPW // CH

_INF = 3.0e38


def _stage1_body(pos_ref, post_ref, x_ref, wkT_ref, wvT_ref, a1T_ref,
                 tab_ref, idx_ref, pos4_ref):
    b = pl.program_id(0)
    pb = pos_ref[0]
    pT = post_ref[0]
    xb = x_ref[0]

    xk = jnp.dot(xb, wkT_ref[...], preferred_element_type=jnp.float32)
    tab_ref[:, 0:D] = jnp.dot(xk, a1T_ref[...], preferred_element_type=jnp.float32)
    tab_ref[:, D:TW] = jnp.dot(xb, wvT_ref[...], preferred_element_type=jnp.float32)

    sqi = jnp.sum(pb * pb, axis=1, keepdims=True)
    sqj = jnp.sum(pT * pT, axis=0, keepdims=True)
    pbh = pb.astype(jnp.bfloat16).astype(jnp.float32)
    pTh = pT.astype(jnp.bfloat16).astype(jnp.float32)
    dot = (pbh[:, 0:1] * pTh[0:1, :]
           + pbh[:, 1:2] * pTh[1:2, :]
           + pbh[:, 2:3] * pTh[2:3, :])
    dw = (sqi + sqj) - 2.0 * dot

    iota = lax.broadcasted_iota(jnp.int32, (BN1, N), 1)
    cols = []
    for _ in range(K):
        sel = jnp.argmin(dw, axis=1).astype(jnp.int32).reshape(BN1, 1)
        dw = jnp.where(iota == sel, _INF, dw)
        cols.append(sel.astype(jnp.float32))
    gi = jnp.concatenate(cols, axis=1)
    idx_ref[...] = jnp.transpose(gi, (1, 0)).astype(jnp.int32) + b * N
    pos4_ref[...] = jnp.concatenate(
        [pb, jnp.zeros((BN1, 1), jnp.float32)], axis=1)


def _stage3_body(x_ref, pos_ref, g_ref, p_ref, wqT_ref, pw1T_ref, pb1_ref,
                 pg_ref, pbeta_ref, pw2T_ref, pb2_ref, a1T_ref, ab1_ref,
                 ag_ref, abeta_ref, a2T_ref, ab2_ref, o_ref):
    f32 = jnp.float32
    xb = x_ref[...]
    pb = pos_ref[...][:, 0:3]
    a1T = a1T_ref[...]
    a2T = a2T_ref[...]
    pw2T = pw2T_ref[...]
    qa = jnp.dot(xb, jnp.dot(wqT_ref[...], a1T, preferred_element_type=f32),
                 preferred_element_type=f32)
    pw2a = jnp.dot(pw2T, a1T, preferred_element_type=f32)
    cb = jnp.dot(pb2_ref[...], a1T, preferred_element_type=f32) + ab1_ref[...]
    w1h = pw1T_ref[...].astype(jnp.bfloat16).astype(f32)
    M12 = jnp.full((HID, HID), 1.0 / HID, f32)
    M64 = jnp.full((D, D), 1.0 / D, f32)

    GF = g_ref[...].reshape(K * BN3, TW)
    KA = GF[:, 0:D]
    V = GF[:, D:TW]
    PF = p_ref[...].reshape(K * BN3, 4)
    PBT = jnp.concatenate([pb] * K, axis=0)
    QAT = jnp.concatenate([qa] * K, axis=0)
    pr = PBT - PF[:, 0:3]
    prh = pr.astype(jnp.bfloat16).astype(f32)
    H = (prh[:, 0:1] * w1h[0:1, :]
         + prh[:, 1:2] * w1h[1:2, :]
         + prh[:, 2:3] * w1h[2:3, :]) + pb1_ref[...]
    MU = jnp.dot(H, M12, preferred_element_type=f32)
    T = H - MU
    VAR = jnp.dot(T * T, M12, preferred_element_type=f32)
    HN = T * lax.rsqrt(VAR + EPS) * pg_ref[...] + pbeta_ref[...]
    RL = jnp.maximum(HN, 0.0)
    PE = jnp.dot(RL, pw2T, preferred_element_type=f32) + pb2_ref[...]
    W1L = QAT - KA + jnp.dot(RL, pw2a, preferred_element_type=f32) + cb
    MU2 = jnp.dot(W1L, M64, preferred_element_type=f32)
    T2 = W1L - MU2
    VAR2 = jnp.dot(T2 * T2, M64, preferred_element_type=f32)
    WN = T2 * lax.rsqrt(VAR2 + EPS) * ag_ref[...] + abeta_ref[...]
    WR = jnp.maximum(WN, 0.0)
    LOGIT = jnp.dot(WR, a2T, preferred_element_type=f32)
    VAL = V + PE

    L3 = LOGIT.reshape(K, BN3, D)
    V3 = VAL.reshape(K, BN3, D)
    m = L3[0]
    for k in range(1, K):
        m = jnp.maximum(m, L3[k])
    s = None
    acc = None
    for k in range(K):
        e = jnp.exp(L3[k] - m)
        s = e if s is None else s + e
        ev = e * V3[k]
        acc = ev if acc is None else acc + ev
    o_ref[...] = acc / s


def _sc_gather_body(tab_hbm, gidx_hbm, pos4_hbm, out_hbm, pout_hbm,
                    idx0_v, idx1_v, rows0_v, rows1_v, ptab_v, prow0_v,
                    prow1_v, sem0, sem1):
    wid = lax.axis_index("s") * NC + lax.axis_index("c")
    pltpu.sync_copy(pos4_hbm, ptab_v)
    lane = lax.broadcasted_iota(jnp.int32, (L,), 0)

    def pos_gather(idx_v, prow_v):
        for i in range(CH // L):
            iv = idx_v[pl.ds(i * L, L)] * 4
            for c in range(3):
                px = plsc.load_gather(ptab_v, [iv + c])
                plsc.store_scatter(prow_v, [lane * 4 + (i * 4 * L + c)], px)

    def body(u, carry):
        t0 = 2 * u
        t1 = t0 + 1
        b0 = wid * RPW + t0 * CH
        b1 = wid * RPW + t1 * CH
        pltpu.sync_copy(gidx_hbm.at[pl.ds(b0, CH)], idx0_v)
        cp0 = pltpu.async_copy(tab_hbm.at[idx0_v], rows0_v, sem0)
        pltpu.sync_copy(gidx_hbm.at[pl.ds(b1, CH)], idx1_v)
        cp1 = pltpu.async_copy(tab_hbm.at[idx1_v], rows1_v, sem1)
        pos_gather(idx0_v, prow0_v)
        cp0.wait()
        pltpu.sync_copy(rows0_v, out_hbm.at[pl.ds(b0, CH)])
        pltpu.sync_copy(prow0_v, pout_hbm.at[pl.ds(b0 * 4, CH * 4)])
        pos_gather(idx1_v, prow1_v)
        cp1.wait()
        pltpu.sync_copy(rows1_v, out_hbm.at[pl.ds(b1, CH)])
        pltpu.sync_copy(prow1_v, pout_hbm.at[pl.ds(b1 * 4, CH * 4)])
        return carry

    lax.fori_loop(0, NCH // 2, body, 0)


def _stage1_call(pos, posT, x, WkT, WvT, a1T):
    return pl.pallas_call(
        _stage1_body,
        grid=(B, NB1),
        in_specs=[
            pl.BlockSpec((1, BN1, 3), lambda b, i: (b, i, 0)),
            pl.BlockSpec((1, 3, N), lambda b, i: (b, 0, 0)),
            pl.BlockSpec((1, BN1, D), lambda b, i: (b, i, 0)),
            pl.BlockSpec((D, D), lambda b, i: (0, 0)),
            pl.BlockSpec((D, D), lambda b, i: (0, 0)),
            pl.BlockSpec((D, D), lambda b, i: (0, 0)),
        ],
        out_specs=[
            pl.BlockSpec((BN1, TW), lambda b, i: (b * NB1 + i, 0)),
            pl.BlockSpec((K, BN1), lambda b, i: (0, b * NB1 + i)),
            pl.BlockSpec((BN1, 4), lambda b, i: (b * NB1 + i, 0)),
        ],
        out_shape=[
            jax.ShapeDtypeStruct((B * N, TW), jnp.float32),
            jax.ShapeDtypeStruct((K, B * N), jnp.int32),
            jax.ShapeDtypeStruct((B * N, 4), jnp.float32),
        ],
    )(pos, posT, x, WkT, WvT, a1T)


def _stage2_call(tab, gidx_kmajor, pos4):
    run = pl.kernel(
        _sc_gather_body,
        out_type=(jax.ShapeDtypeStruct((R, TW), jnp.float32),
                  jax.ShapeDtypeStruct((R * 4,), jnp.float32)),
        mesh=plsc.VectorSubcoreMesh(core_axis_name="c", subcore_axis_name="s",
                                    num_cores=NC, num_subcores=NS),
        scratch_types=[
            pltpu.VMEM((CH,), jnp.int32),
            pltpu.VMEM((CH,), jnp.int32),
            pltpu.VMEM((CH, TW), jnp.float32),
            pltpu.VMEM((CH, TW), jnp.float32),
            pltpu.VMEM((B * N * 4,), jnp.float32),
            pltpu.VMEM((CH * 4,), jnp.float32),
            pltpu.VMEM((CH * 4,), jnp.float32),
            pltpu.SemaphoreType.DMA,
            pltpu.SemaphoreType.DMA,
        ],
        compiler_params=pltpu.CompilerParams(needs_layout_passes=False),
    )
    return run(tab, gidx_kmajor, pos4)


def _stage3_call(x2, pos2, G, P, WqT, pw1T, pb1, pg, pbeta, pw2T, pb2,
                 a1T, ab1, ag, abeta, a2T, ab2):
    nsteps = (B * N) // BN3
    full = lambda r, c: pl.BlockSpec((r, c), lambda i: (0, 0))
    return pl.pallas_call(
        _stage3_body,
        grid=(nsteps,),
        in_specs=[
            pl.BlockSpec((BN3, D), lambda i: (i, 0)),
            pl.BlockSpec((BN3, 4), lambda i: (i, 0)),
            pl.BlockSpec((K, BN3, TW), lambda i: (0, i, 0)),
            pl.BlockSpec((K, BN3, 4), lambda i: (0, i, 0)),
            full(D, D), full(3, HID), full(1, HID), full(1, HID),
            full(1, HID), full(HID, D), full(1, D), full(D, D), full(1, D),
            full(1, D), full(1, D), full(D, D), full(1, D),
        ],
        out_specs=pl.BlockSpec((BN3, D), lambda i: (i, 0)),
        out_shape=jax.ShapeDtypeStruct((B * N, D), jnp.float32),
    )(x2, pos2, G, P, WqT, pw1T, pb1, pg, pbeta, pw2T, pb2,
      a1T, ab1, ag, abeta, a2T, ab2)


def kernel(x, pos, Wq, Wk, Wv, pos_w1, pos_b1, pos_g, pos_beta, pos_w2,
           pos_b2, attn_w1, attn_b1, attn_g, attn_beta, attn_w2, attn_b2):
    posT = jnp.swapaxes(pos, 1, 2)
    tab, gidx, pos42d = _stage1_call(pos, posT, x, Wk.T, Wv.T, attn_w1.T)
    gidx_kmajor = gidx.reshape(R)
    G, P = _stage2_call(tab, gidx_kmajor, pos42d.reshape(B * N * 4))
    G = G.reshape(K, B * N, TW)
    P = P.reshape(K, B * N, 4)
    agg = _stage3_call(
        x.reshape(B * N, D), pos42d, G, P,
        Wq.T, pos_w1.T, pos_b1.reshape(1, HID), pos_g.reshape(1, HID),
        pos_beta.reshape(1, HID), pos_w2.T, pos_b2.reshape(1, D),
        attn_w1.T, attn_b1.reshape(1, D), attn_g.reshape(1, D),
        attn_beta.reshape(1, D), attn_w2.T, attn_b2.reshape(1, D))
    return agg.reshape(B, N, D)
